# baseline (speedup 1.0000x reference)
import jax
import jax.numpy as jnp
from jax import lax
from jax.experimental import pallas as pl
from jax.experimental.pallas import tpu as pltpu
from jax.experimental.pallas import tpu_sc as plsc

MAX_THETA = 90.0
MAX_RADIUS = 400.0
TH_THETA = 5.0
TH_RADIUS = 20.0
W_CLS = 1.0
W_REG = 1.0

_B, _N, _G = 16, 4096, 32
_L = 16
_C_TH = TH_THETA / MAX_THETA
_C_RA = TH_RADIUS / MAX_RADIUS
_S_TH = 1.0 / (2.0 * MAX_THETA)
_S_RA = 1.0 / (2.0 * MAX_RADIUS)

_MESH = plsc.VectorSubcoreMesh(core_axis_name="c", subcore_axis_name="s",
                               num_cores=2, num_subcores=16)


def _lane_bcast(vec, lane):
    idx = jnp.full((_L,), lane, dtype=jnp.int32)
    return vec.at[idx].get(mode="promise_in_bounds")


def _sc_match_body(pith_h, pira_h, pth_h, pra_h, tt_h, tr_h,
                   hp_out, fb_out, piece_out,
                   pith_v, pira_v, pth_v, pra_v, tt_v, tr_v,
                   hp_v, fb_v, piece_v):
    wid = lax.axis_index("s") * 2 + lax.axis_index("c")
    b = wid // 4
    gq = wid % 4
    pltpu.sync_copy(pith_h.at[b], pith_v)
    pltpu.sync_copy(pira_h.at[b], pira_v)
    pltpu.sync_copy(pth_h.at[b], pth_v)
    pltpu.sync_copy(pra_h.at[b], pra_v)
    pltpu.sync_copy(tt_h.at[b], tt_v)
    pltpu.sync_copy(tr_h.at[b], tr_v)

    lane = lax.iota(jnp.int32, _L)
    c_th2 = jnp.float32(_C_TH) * jnp.float32(_C_TH)
    c_ra2 = jnp.float32(_C_RA) * jnp.float32(_C_RA)

    for t in range(8):
        g = gq * 8 + t
        gsplat = jnp.full((_L,), 0, dtype=jnp.int32) + g
        ttb = (plsc.load_gather(tt_v, [gsplat]) + MAX_THETA) * _S_TH
        trb = (plsc.load_gather(tr_v, [gsplat]) + MAX_RADIUS) * _S_RA

        def body(i, carry, ttb=ttb, trb=trb):
            minv, idxv, anyv = carry
            for u in range(8):
                base = (i * 8 + u) * _L
                vth = pith_v[pl.ds(base, _L)]
                vra = pira_v[pl.ds(base, _L)]
                tds = ttb - vth
                rds = trb - vra
                td2 = tds * tds
                rd2 = rds * rds
                cf = jnp.where((td2 < c_th2) & (rd2 < c_ra2), 1.0, 0.0)
                anyv = jnp.maximum(anyv, cf)
                d2 = td2 + rd2
                upd = d2 < minv
                minv = jnp.where(upd, d2, minv)
                idxv = jnp.where(upd, base + lane, idxv)
            return (minv, idxv, anyv)

        init = (jnp.full((_L,), jnp.inf, jnp.float32),
                jnp.zeros((_L,), jnp.int32),
                jnp.zeros((_L,), jnp.float32))
        minv, idxv, anyv = lax.fori_loop(0, _N // (_L * 8), body, init)

        hp_s = jnp.max(anyv)
        m_s = jnp.min(minv)
        fb_s = jnp.min(jnp.where(minv == m_s, idxv, _N))
        fbi = jnp.full((_L,), 0, dtype=jnp.int32) + fb_s
        pt_fb = plsc.load_gather(pth_v, [fbi])
        pr_fb = plsc.load_gather(pra_v, [fbi])
        dtf = ttb - pt_fb
        drf = trb - pr_fb
        costfb2 = dtf * dtf + drf * drf
        sel = lane == t
        hp_v[...] = jnp.where(sel, hp_s, hp_v[...])
        fb_v[...] = jnp.where(sel, fb_s, fb_v[...])
        piece_v[...] = jnp.where(sel, costfb2, piece_v[...])

    pltpu.sync_copy(hp_v.at[pl.ds(0, 8)], hp_out.at[b, pl.ds(gq * 8, 8)])
    pltpu.sync_copy(fb_v.at[pl.ds(0, 8)], fb_out.at[b, pl.ds(gq * 8, 8)])
    pltpu.sync_copy(piece_v.at[pl.ds(0, 8)],
                    piece_out.at[b, pl.ds(gq * 8, 8)])


def _tc_prelude_body(l0, l1, pith, pira, pth, pra, tt, tr, tp0,
                     bits_out, csum_out, hp_out, fb_out, cfb_out,
                     d_out, s_out):
    B, N = l0.shape
    G = tt.shape[1]
    H = B // 2
    pith_v = pith[...]
    pira_v = pira[...]
    pth_v = pth[...]
    pra_v = pra[...]
    ttn = (tt[...] + MAX_THETA) * _S_TH
    trn = (tr[...] + MAX_RADIUS) * _S_RA
    valid = tp0[...] != -1000.0
    iota8 = jax.lax.broadcasted_iota(jnp.int32, (H, N), 1)

    bits = jnp.zeros((B, N), dtype=jnp.int32)
    csums = []
    hps = []
    fbs = []
    cfbs = []
    for g in range(G):
        ttg = ttn[:, g:g + 1]
        trg = trn[:, g:g + 1]
        vg = valid[:, g:g + 1]
        td = jnp.abs(ttg - pith_v)
        rd = jnp.abs(trg - pira_v)
        cond = (td < _C_TH) & (rd < _C_RA)
        dt = ttg - pth_v
        dr = trg - pra_v
        cost2 = dt * dt + dr * dr
        csums.append(jnp.sum(jnp.where(cond, cost2, 0.0), axis=1,
                             keepdims=True))
        gb = (1 << g) if g < 31 else -(1 << 31)
        bits = bits | jnp.where(cond & vg, jnp.int32(gb), jnp.int32(0))
        td8 = td[H:, :]
        rd8 = rd[H:, :]
        cf8 = jnp.where(cond[H:, :], 1.0, 0.0)
        hps.append(jnp.max(cf8, axis=1, keepdims=True))
        d2 = td8 * td8 + rd8 * rd8
        m = jnp.min(d2, axis=1, keepdims=True)
        fb = jnp.min(jnp.where(d2 == m, iota8, N), axis=1, keepdims=True)
        fbs.append(fb)
        c28 = cost2[H:, :]
        cfbs.append(jnp.sum(jnp.where(iota8 == fb, c28, 0.0), axis=1,
                            keepdims=True))
    bits_out[...] = bits
    csum_out[...] = jnp.concatenate(csums, axis=1)
    hp_out[...] = jnp.concatenate(hps, axis=1)
    fb_out[...] = jnp.concatenate(fbs, axis=1)
    cfb_out[...] = jnp.concatenate(cfbs, axis=1)

    l0_v = l0[...]
    l1_v = l1[...]
    mx = jnp.maximum(l0_v, l1_v)
    a0 = l0_v - mx
    a1 = l1_v - mx
    e0 = jnp.exp(a0)
    e1 = jnp.exp(a1)
    z = e0 + e1
    logz = jnp.log(z)
    s0 = e0 / z
    s1 = e1 / z
    t0 = (s1 * s1) * (a0 - logz)
    t1 = (s0 * s0) * (a1 - logz)
    d_out[...] = t1 - t0
    s_out[0, 0] = jnp.sum(t0)


def _tc_combine_body(bits, d_in, s_in, tp0, hp_sc, fb_sc, cfb_sc,
                     hp_tc, fb_tc, cfb_tc, csum2, out):
    B, N = bits.shape
    G = hp_tc.shape[1]
    hp = jnp.concatenate([hp_sc[...], hp_tc[...]], axis=0)
    fb = jnp.concatenate([fb_sc[...], fb_tc[...]], axis=0)
    costfb2 = jnp.concatenate([cfb_sc[...], cfb_tc[...]], axis=0)
    iota_n = jax.lax.broadcasted_iota(jnp.int32, (B, N), 1)
    valid = tp0[...] != -1000.0
    hp_b = hp > 0.5
    gbit = (jnp.int32(1) << jax.lax.broadcasted_iota(jnp.int32, (B, G), 1))
    hpm = jnp.sum(jnp.where(hp_b, gbit, jnp.int32(0)), axis=1, keepdims=True)
    gt = (bits[...] & hpm) != 0
    fbx = jnp.where(valid & (~hp_b), fb, -1)
    for g in range(G):
        gt = gt | (iota_n == fbx[:, g:g + 1])
    picked_sum = s_in[0, 0] + jnp.sum(jnp.where(gt, d_in[...], 0.0))
    loss_cls = -picked_sum / (B * N)
    piece = jnp.where(valid, jnp.where(hp_b, csum2[...], costfb2), 0.0)
    loss_reg = (0.5 * jnp.sum(piece)) / _B
    out[0, 0] = W_CLS * loss_cls + W_REG * loss_reg


@jax.jit
def _run(l0, l1, pth, pra, pith, pira, tt, tr, tp0):
    f32 = jnp.float32
    match = pl.kernel(
        _sc_match_body,
        out_type=(jax.ShapeDtypeStruct((_B // 2, _G), f32),
                  jax.ShapeDtypeStruct((_B // 2, _G), jnp.int32),
                  jax.ShapeDtypeStruct((_B // 2, _G), f32)),
        mesh=_MESH,
        compiler_params=pltpu.CompilerParams(needs_layout_passes=False),
        scratch_types=[
            pltpu.VMEM((_N,), f32), pltpu.VMEM((_N,), f32),
            pltpu.VMEM((_N,), f32), pltpu.VMEM((_N,), f32),
            pltpu.VMEM((_G,), f32), pltpu.VMEM((_G,), f32),
            pltpu.VMEM((_L,), f32), pltpu.VMEM((_L,), jnp.int32),
            pltpu.VMEM((_L,), f32),
        ],
    )
    hp_sc, fb_sc, cfb_sc = match(pith, pira, pth, pra, tt, tr)

    bits, csum2, hp_tc, fb_tc, cfb_tc, d_arr, s_arr = pl.pallas_call(
        _tc_prelude_body,
        out_shape=(jax.ShapeDtypeStruct((_B, _N), jnp.int32),
                   jax.ShapeDtypeStruct((_B, _G), f32),
                   jax.ShapeDtypeStruct((_B // 2, _G), f32),
                   jax.ShapeDtypeStruct((_B // 2, _G), jnp.int32),
                   jax.ShapeDtypeStruct((_B // 2, _G), f32),
                   jax.ShapeDtypeStruct((_B, _N), f32),
                   jax.ShapeDtypeStruct((1, 1), f32)),
        in_specs=[pl.BlockSpec(memory_space=pltpu.VMEM) for _ in range(9)],
        out_specs=(pl.BlockSpec(memory_space=pltpu.VMEM),
                   pl.BlockSpec(memory_space=pltpu.VMEM),
                   pl.BlockSpec(memory_space=pltpu.VMEM),
                   pl.BlockSpec(memory_space=pltpu.VMEM),
                   pl.BlockSpec(memory_space=pltpu.VMEM),
                   pl.BlockSpec(memory_space=pltpu.VMEM),
                   pl.BlockSpec(memory_space=pltpu.SMEM)),
    )(l0, l1, pith, pira, pth, pra, tt, tr, tp0)

    out = pl.pallas_call(
        _tc_combine_body,
        out_shape=jax.ShapeDtypeStruct((1, 1), f32),
        in_specs=[pl.BlockSpec(memory_space=pltpu.VMEM),
                  pl.BlockSpec(memory_space=pltpu.VMEM),
                  pl.BlockSpec(memory_space=pltpu.SMEM)]
        + [pl.BlockSpec(memory_space=pltpu.VMEM) for _ in range(8)],
        out_specs=pl.BlockSpec(memory_space=pltpu.SMEM),
    )(bits, d_arr, s_arr, tp0, hp_sc, fb_sc, cfb_sc,
      hp_tc, fb_tc, cfb_tc, csum2)
    return out[0, 0]


def kernel(cls_logits, params, params_init, tgt_params, tgt_pts):
    return _run(cls_logits[:, :, 0], cls_logits[:, :, 1],
                params[:, :, 0], params[:, :, 1],
                params_init[:, :, 0], params_init[:, :, 1],
                tgt_params[:, :, 0], tgt_params[:, :, 1],
                tgt_pts[:, :, 0])


# speedup vs baseline: 1.0054x; 1.0054x over previous
import jax
import jax.numpy as jnp
from jax import lax
from jax.experimental import pallas as pl
from jax.experimental.pallas import tpu as pltpu
from jax.experimental.pallas import tpu_sc as plsc

MAX_THETA = 90.0
MAX_RADIUS = 400.0
TH_THETA = 5.0
TH_RADIUS = 20.0
W_CLS = 1.0
W_REG = 1.0

_B, _N, _G = 16, 4096, 32
_L = 16
_C_TH = TH_THETA / MAX_THETA
_C_RA = TH_RADIUS / MAX_RADIUS
_S_TH = 1.0 / (2.0 * MAX_THETA)
_S_RA = 1.0 / (2.0 * MAX_RADIUS)

_MESH = plsc.VectorSubcoreMesh(core_axis_name="c", subcore_axis_name="s",
                               num_cores=2, num_subcores=16)


def _lane_bcast(vec, lane):
    idx = jnp.full((_L,), lane, dtype=jnp.int32)
    return vec.at[idx].get(mode="promise_in_bounds")


def _sc_match_body(pith_h, pira_h, pth_h, pra_h, tt_h, tr_h,
                   hp_out, fb_out, piece_out,
                   pith_v, pira_v, pth_v, pra_v, tt_v, tr_v,
                   hp_v, fb_v, piece_v):
    wid = lax.axis_index("s") * 2 + lax.axis_index("c")
    b = wid // 4
    gq = wid % 4
    pltpu.sync_copy(pith_h.at[b], pith_v)
    pltpu.sync_copy(pira_h.at[b], pira_v)
    pltpu.sync_copy(pth_h.at[b], pth_v)
    pltpu.sync_copy(pra_h.at[b], pra_v)
    pltpu.sync_copy(tt_h.at[b], tt_v)
    pltpu.sync_copy(tr_h.at[b], tr_v)

    lane = lax.iota(jnp.int32, _L)
    c_th2 = jnp.float32(_C_TH) * jnp.float32(_C_TH)
    c_ra2 = jnp.float32(_C_RA) * jnp.float32(_C_RA)

    for t in range(8):
        g = gq * 8 + t
        gsplat = jnp.full((_L,), 0, dtype=jnp.int32) + g
        ttb = (plsc.load_gather(tt_v, [gsplat]) + MAX_THETA) * _S_TH
        trb = (plsc.load_gather(tr_v, [gsplat]) + MAX_RADIUS) * _S_RA

        def body(i, carry, ttb=ttb, trb=trb):
            minv, idxv, anyv = carry
            for u in range(8):
                base = (i * 8 + u) * _L
                vth = pith_v[pl.ds(base, _L)]
                vra = pira_v[pl.ds(base, _L)]
                tds = ttb - vth
                rds = trb - vra
                td2 = tds * tds
                rd2 = rds * rds
                cf = jnp.where((td2 < c_th2) & (rd2 < c_ra2), 1.0, 0.0)
                anyv = jnp.maximum(anyv, cf)
                d2 = td2 + rd2
                upd = d2 < minv
                minv = jnp.where(upd, d2, minv)
                idxv = jnp.where(upd, base + lane, idxv)
            return (minv, idxv, anyv)

        init = (jnp.full((_L,), jnp.inf, jnp.float32),
                jnp.zeros((_L,), jnp.int32),
                jnp.zeros((_L,), jnp.float32))
        minv, idxv, anyv = lax.fori_loop(0, _N // (_L * 8), body, init)

        hp_s = jnp.max(anyv)
        m_s = jnp.min(minv)
        fb_s = jnp.min(jnp.where(minv == m_s, idxv, _N))
        fbi = jnp.full((_L,), 0, dtype=jnp.int32) + fb_s
        pt_fb = plsc.load_gather(pth_v, [fbi])
        pr_fb = plsc.load_gather(pra_v, [fbi])
        dtf = ttb - pt_fb
        drf = trb - pr_fb
        costfb2 = dtf * dtf + drf * drf
        sel = lane == t
        hp_v[...] = jnp.where(sel, hp_s, hp_v[...])
        fb_v[...] = jnp.where(sel, fb_s, fb_v[...])
        piece_v[...] = jnp.where(sel, costfb2, piece_v[...])

    pltpu.sync_copy(hp_v.at[pl.ds(0, 8)], hp_out.at[b, pl.ds(gq * 8, 8)])
    pltpu.sync_copy(fb_v.at[pl.ds(0, 8)], fb_out.at[b, pl.ds(gq * 8, 8)])
    pltpu.sync_copy(piece_v.at[pl.ds(0, 8)],
                    piece_out.at[b, pl.ds(gq * 8, 8)])


def _tc_prelude_body(l0, l1, pith, pira, pth, pra, tt, tr, tp0,
                     bits_out, csum_out, hp_out, fb_out, cfb_out,
                     d_out, s_out):
    B, N = l0.shape
    G = tt.shape[1]
    H = B // 2
    pith_v = pith[...]
    pira_v = pira[...]
    pth_v = pth[...]
    pra_v = pra[...]
    ttn = (tt[...] + MAX_THETA) * _S_TH
    trn = (tr[...] + MAX_RADIUS) * _S_RA
    valid = tp0[...] != -1000.0
    iota8 = jax.lax.broadcasted_iota(jnp.int32, (H, N), 1)
    c_th2 = jnp.float32(_C_TH) * jnp.float32(_C_TH)
    c_ra2 = jnp.float32(_C_RA) * jnp.float32(_C_RA)

    bits = jnp.zeros((B, N), dtype=jnp.int32)
    csums = []
    hps = []
    fbs = []
    cfbs = []
    for g in range(G):
        ttg = ttn[:, g:g + 1]
        trg = trn[:, g:g + 1]
        vg = valid[:, g:g + 1]
        tds = ttg - pith_v
        rds = trg - pira_v
        td2 = tds * tds
        rd2 = rds * rds
        cond = (td2 < c_th2) & (rd2 < c_ra2)
        dt = ttg - pth_v
        dr = trg - pra_v
        cost2 = dt * dt + dr * dr
        csums.append(jnp.sum(jnp.where(cond, cost2, 0.0), axis=1,
                             keepdims=True))
        gb = (1 << g) if g < 31 else -(1 << 31)
        bits = bits | jnp.where(cond & vg, jnp.int32(gb), jnp.int32(0))
        cf8 = jnp.where(cond[H:, :], 1.0, 0.0)
        hps.append(jnp.max(cf8, axis=1, keepdims=True))
        d2 = td2[H:, :] + rd2[H:, :]
        m = jnp.min(d2, axis=1, keepdims=True)
        fb = jnp.min(jnp.where(d2 == m, iota8, N), axis=1, keepdims=True)
        fbs.append(fb)
        c28 = cost2[H:, :]
        cfbs.append(jnp.sum(jnp.where(iota8 == fb, c28, 0.0), axis=1,
                            keepdims=True))
    bits_out[...] = bits
    csum_out[...] = jnp.concatenate(csums, axis=1)
    hp_out[...] = jnp.concatenate(hps, axis=1)
    fb_out[...] = jnp.concatenate(fbs, axis=1)
    cfb_out[...] = jnp.concatenate(cfbs, axis=1)

    l0_v = l0[...]
    l1_v = l1[...]
    mx = jnp.maximum(l0_v, l1_v)
    a0 = l0_v - mx
    a1 = l1_v - mx
    e0 = jnp.exp(a0)
    e1 = jnp.exp(a1)
    z = e0 + e1
    logz = jnp.log(z)
    s0 = e0 / z
    s1 = e1 / z
    t0 = (s1 * s1) * (a0 - logz)
    t1 = (s0 * s0) * (a1 - logz)
    d_out[...] = t1 - t0
    s_out[0, 0] = jnp.sum(t0)


def _tc_combine_body(bits, d_in, s_in, tp0, hp_sc, fb_sc, cfb_sc,
                     hp_tc, fb_tc, cfb_tc, csum2, out):
    B, N = bits.shape
    G = hp_tc.shape[1]
    hp = jnp.concatenate([hp_sc[...], hp_tc[...]], axis=0)
    fb = jnp.concatenate([fb_sc[...], fb_tc[...]], axis=0)
    costfb2 = jnp.concatenate([cfb_sc[...], cfb_tc[...]], axis=0)
    iota_n = jax.lax.broadcasted_iota(jnp.int32, (B, N), 1)
    valid = tp0[...] != -1000.0
    hp_b = hp > 0.5
    gbit = (jnp.int32(1) << jax.lax.broadcasted_iota(jnp.int32, (B, G), 1))
    hpm = jnp.sum(jnp.where(hp_b, gbit, jnp.int32(0)), axis=1, keepdims=True)
    gt = (bits[...] & hpm) != 0
    fbx = jnp.where(valid & (~hp_b), fb, -1)
    for g in range(G):
        gt = gt | (iota_n == fbx[:, g:g + 1])
    picked_sum = s_in[0, 0] + jnp.sum(jnp.where(gt, d_in[...], 0.0))
    loss_cls = -picked_sum / (B * N)
    piece = jnp.where(valid, jnp.where(hp_b, csum2[...], costfb2), 0.0)
    loss_reg = (0.5 * jnp.sum(piece)) / _B
    out[0, 0] = W_CLS * loss_cls + W_REG * loss_reg


@jax.jit
def _run(l0, l1, pth, pra, pith, pira, tt, tr, tp0):
    f32 = jnp.float32
    match = pl.kernel(
        _sc_match_body,
        out_type=(jax.ShapeDtypeStruct((_B // 2, _G), f32),
                  jax.ShapeDtypeStruct((_B // 2, _G), jnp.int32),
                  jax.ShapeDtypeStruct((_B // 2, _G), f32)),
        mesh=_MESH,
        compiler_params=pltpu.CompilerParams(needs_layout_passes=False),
        scratch_types=[
            pltpu.VMEM((_N,), f32), pltpu.VMEM((_N,), f32),
            pltpu.VMEM((_N,), f32), pltpu.VMEM((_N,), f32),
            pltpu.VMEM((_G,), f32), pltpu.VMEM((_G,), f32),
            pltpu.VMEM((_L,), f32), pltpu.VMEM((_L,), jnp.int32),
            pltpu.VMEM((_L,), f32),
        ],
    )
    hp_sc, fb_sc, cfb_sc = match(pith, pira, pth, pra, tt, tr)

    bits, csum2, hp_tc, fb_tc, cfb_tc, d_arr, s_arr = pl.pallas_call(
        _tc_prelude_body,
        out_shape=(jax.ShapeDtypeStruct((_B, _N), jnp.int32),
                   jax.ShapeDtypeStruct((_B, _G), f32),
                   jax.ShapeDtypeStruct((_B // 2, _G), f32),
                   jax.ShapeDtypeStruct((_B // 2, _G), jnp.int32),
                   jax.ShapeDtypeStruct((_B // 2, _G), f32),
                   jax.ShapeDtypeStruct((_B, _N), f32),
                   jax.ShapeDtypeStruct((1, 1), f32)),
        in_specs=[pl.BlockSpec(memory_space=pltpu.VMEM) for _ in range(9)],
        out_specs=(pl.BlockSpec(memory_space=pltpu.VMEM),
                   pl.BlockSpec(memory_space=pltpu.VMEM),
                   pl.BlockSpec(memory_space=pltpu.VMEM),
                   pl.BlockSpec(memory_space=pltpu.VMEM),
                   pl.BlockSpec(memory_space=pltpu.VMEM),
                   pl.BlockSpec(memory_space=pltpu.VMEM),
                   pl.BlockSpec(memory_space=pltpu.SMEM)),
    )(l0, l1, pith, pira, pth, pra, tt, tr, tp0)

    out = pl.pallas_call(
        _tc_combine_body,
        out_shape=jax.ShapeDtypeStruct((1, 1), f32),
        in_specs=[pl.BlockSpec(memory_space=pltpu.VMEM),
                  pl.BlockSpec(memory_space=pltpu.VMEM),
                  pl.BlockSpec(memory_space=pltpu.SMEM)]
        + [pl.BlockSpec(memory_space=pltpu.VMEM) for _ in range(8)],
        out_specs=pl.BlockSpec(memory_space=pltpu.SMEM),
    )(bits, d_arr, s_arr, tp0, hp_sc, fb_sc, cfb_sc,
      hp_tc, fb_tc, cfb_tc, csum2)
    return out[0, 0]


def kernel(cls_logits, params, params_init, tgt_params, tgt_pts):
    return _run(cls_logits[:, :, 0], cls_logits[:, :, 1],
                params[:, :, 0], params[:, :, 1],
                params_init[:, :, 0], params_init[:, :, 1],
                tgt_params[:, :, 0], tgt_params[:, :, 1],
                tgt_pts[:, :, 0])
